# baseline (speedup 1.0000x reference)
import functools

import jax
import jax.numpy as jnp
import numpy as np
from jax import lax
from jax.experimental import pallas as pl
from jax.experimental.pallas import tpu as pltpu
from jax.experimental.pallas import tpu_sc as plsc

N = 10000
E = 320000
D_IN = 128
D_HID = 128
D_OUT = 64

NUM_CORES = 2
NUM_SUBCORES = 16
NUM_TILES = NUM_CORES * NUM_SUBCORES
EPT = E // NUM_TILES
K = 80
NCH = EPT // K
RPT = N // NUM_SUBCORES
ZR = 125
WB = 624
WB_TAIL = N - NUM_SUBCORES * WB
NBUF = 5

_MESH = plsc.VectorSubcoreMesh(core_axis_name="c", subcore_axis_name="s")
_SC_PARAMS = pltpu.CompilerParams(use_tc_tiling_on_sc=False,
                                  needs_layout_passes=False)

_Q64 = np.empty(64, np.int64)
for _h in range(2):
    for _u in range(16):
        _Q64[32 * _h + _u] = 32 * _h + 2 * _u
        _Q64[32 * _h + 16 + _u] = 32 * _h + 2 * _u + 1
_P64 = np.zeros((64, 64), np.float32)
_P64[np.arange(64), _Q64] = 1.0


def _zero_acc(zbuf, acc, sid, d):
    def _zrow(i, carry):
        for j in range(d // 16):
            zbuf[i, pl.ds(j * 16, 16)] = jnp.zeros((16,), jnp.float32)
        return carry

    lax.fori_loop(0, ZR, _zrow, 0)
    for z in range(RPT // ZR):
        pltpu.sync_copy(zbuf, acc.at[pl.ds(sid * RPT + z * ZR, ZR)])


def _deg_body(col_hbm, ew_hbm, out_hbm, cidx, ewv, rows, zbuf, acc, sem):
    cid = lax.axis_index("c")
    sid = lax.axis_index("s")
    wid = cid * NUM_SUBCORES + sid

    _zero_acc(zbuf, acc, sid, 16)
    plsc.subcore_barrier()

    pltpu.sync_copy(col_hbm.at[wid], cidx)
    pltpu.sync_copy(ew_hbm.at[wid], ewv)

    def chunk(i, carry):
        def fill(g, c2):
            wv = ewv[i, pl.ds(g * 16, 16)]
            for j in range(16):
                rows[g * 16 + j, :] = jnp.full((16,), wv[j], jnp.float32)
            return c2

        lax.fori_loop(0, K // 16, fill, 0)
        pltpu.sync_copy(rows, acc.at[cidx.at[i]], add=True)
        return carry

    lax.fori_loop(0, NCH, chunk, 0)
    plsc.subcore_barrier()
    pltpu.sync_copy(acc.at[pl.ds(sid * WB, WB)],
                    out_hbm.at[cid, pl.ds(sid * WB, WB)])

    @pl.when(sid == 0)
    def _tail():
        pltpu.sync_copy(acc.at[pl.ds(NUM_SUBCORES * WB, WB_TAIL)],
                        out_hbm.at[cid, pl.ds(NUM_SUBCORES * WB, WB_TAIL)])


_deg_kernel = functools.partial(
    pl.kernel,
    out_type=jax.ShapeDtypeStruct((NUM_CORES, N, 16), jnp.float32),
    mesh=_MESH,
    scratch_types=[
        pltpu.VMEM((NCH, K), jnp.int32),
        pltpu.VMEM((NCH, K), jnp.float32),
        pltpu.VMEM((K, 16), jnp.float32),
        pltpu.VMEM((ZR, 16), jnp.float32),
        pltpu.VMEM_SHARED((N, 16), jnp.float32),
        pltpu.SemaphoreType.DMA,
    ],
    compiler_params=_SC_PARAMS,
)(_deg_body)


def _msg_body(d, y_hbm, row_hbm, col_hbm, ew_hbm, out_hbm,
              ridx, cidx, ewv, bufs, fbufs, zbuf, acc, gsems, ssems):
    cid = lax.axis_index("c")
    sid = lax.axis_index("s")
    wid = cid * NUM_SUBCORES + sid

    _zero_acc(zbuf, acc, sid, d)
    plsc.subcore_barrier()

    pltpu.sync_copy(row_hbm.at[wid], ridx)
    pltpu.sync_copy(col_hbm.at[wid], cidx)
    pltpu.sync_copy(ew_hbm.at[wid], ewv)

    def gather(i, q):
        return pltpu.make_async_copy(y_hbm.at[ridx.at[i]], bufs[q], gsems[q])

    def scatter(i, q):
        return pltpu.make_async_copy(fbufs[q], acc.at[cidx.at[i]], ssems[q])

    def scale(i, q):
        buf = bufs[q]
        fbuf = fbufs[q]

        def body(g, c2):
            wv = ewv[i, pl.ds(g * 16, 16)]
            for u in range(16):
                e = g * 16 + u
                w = wv[u]
                for h in range(d // 32):
                    x32 = buf[e, pl.ds(h * 32, 32)]
                    a, b = plsc.unpack(
                        x32, format=plsc.PackFormat.INTERLEAVED)
                    fbuf[e, pl.ds(h * 32, 16)] = a * w
                    fbuf[e, pl.ds(h * 32 + 16, 16)] = b * w
            return c2

        lax.fori_loop(0, K // 16, body, 0)

    pltpu.make_async_copy(zbuf.at[pl.ds(0, K)], acc.at[cidx.at[0]],
                          ssems[NBUF - 1]).start(add=True)
    for i in range(NBUF - 1):
        gather(i, i).start()

    def block(i, q, refill):
        gather(i, q).wait()
        scale(i, q)
        scatter(i, q).start(add=True)
        scatter(i - 1, (q - 1) % NBUF).wait()
        if refill:
            gather(i + NBUF - 1, (q - 1) % NBUF).start()

    def quad(t, carry):
        i0 = NBUF * t
        for q in range(NBUF):
            block(i0 + q, q, True)
        return carry

    nloop = (NCH - (NBUF + 1)) // NBUF
    lax.fori_loop(0, nloop, quad, 0)
    for i in range(NBUF * nloop, NCH):
        block(i, i % NBUF, i + NBUF - 1 < NCH)
    scatter(NCH - 1, (NCH - 1) % NBUF).wait()
    plsc.subcore_barrier()
    pltpu.sync_copy(acc.at[pl.ds(sid * WB, WB)],
                    out_hbm.at[cid, pl.ds(sid * WB, WB)])

    @pl.when(sid == 0)
    def _tail():
        pltpu.sync_copy(acc.at[pl.ds(NUM_SUBCORES * WB, WB_TAIL)],
                        out_hbm.at[cid, pl.ds(NUM_SUBCORES * WB, WB_TAIL)])


def _make_msg_kernel(d):
    return functools.partial(
        pl.kernel,
        out_type=jax.ShapeDtypeStruct((NUM_CORES, N, d), jnp.float32),
        mesh=_MESH,
        scratch_types=[
            pltpu.VMEM((NCH, K), jnp.int32),
            pltpu.VMEM((NCH, K), jnp.int32),
            pltpu.VMEM((NCH, K), jnp.float32),
            [pltpu.VMEM((K, d), jnp.bfloat16) for _ in range(NBUF)],
            [pltpu.VMEM((K, d), jnp.float32) for _ in range(NBUF)],
            pltpu.VMEM((ZR, d), jnp.float32),
            pltpu.VMEM_SHARED((N, d), jnp.float32),
            [pltpu.SemaphoreType.DMA for _ in range(NBUF)],
            [pltpu.SemaphoreType.DMA for _ in range(NBUF)],
        ],
        compiler_params=_SC_PARAMS,
    )(functools.partial(_msg_body, d))


_msg_kernel_64 = _make_msg_kernel(D_OUT)


_R = 1000
_G = N // _R


def _dinv_of(degp_ref):
    deg = degp_ref[0, :, :1] + degp_ref[1, :, :1] + 1.0
    return lax.rsqrt(deg)


def _mm_body(x_ref, w_ref, degp_ref, xw_ref, ya_ref, yb_ref):
    xw = jnp.dot(x_ref[...], w_ref[...], preferred_element_type=jnp.float32)
    xw_ref[...] = xw
    y = (xw * _dinv_of(degp_ref)).astype(jnp.bfloat16)
    ya_ref[...] = y[:, :D_OUT]
    yb_ref[...] = y[:, D_OUT:]


def _matmul_prescale(x, w, degp):
    return pl.pallas_call(
        _mm_body,
        grid=(_G,),
        in_specs=[
            pl.BlockSpec((_R, x.shape[1]), lambda i: (i, 0)),
            pl.BlockSpec(w.shape, lambda i: (0, 0)),
            pl.BlockSpec((NUM_CORES, _R, 16), lambda i: (0, i, 0)),
        ],
        out_specs=[
            pl.BlockSpec((_R, w.shape[1]), lambda i: (i, 0)),
            pl.BlockSpec((_R, D_OUT), lambda i: (i, 0)),
            pl.BlockSpec((_R, D_OUT), lambda i: (i, 0)),
        ],
        out_shape=[
            jax.ShapeDtypeStruct((x.shape[0], w.shape[1]), jnp.float32),
            jax.ShapeDtypeStruct((N, D_OUT), jnp.bfloat16),
            jax.ShapeDtypeStruct((N, D_OUT), jnp.bfloat16),
        ],
    )(x, w, degp)


def _layer_body(aggpa_ref, aggpb_ref, xw_ref, degp_ref, b_ref, w_ref,
                p_ref, xw2_ref, y2_ref):
    dinv = _dinv_of(degp_ref)
    p = p_ref[...]
    agga = jnp.dot(aggpa_ref[0] + aggpa_ref[1], p,
                   preferred_element_type=jnp.float32)
    aggb = jnp.dot(aggpb_ref[0] + aggpb_ref[1], p,
                   preferred_element_type=jnp.float32)
    agg = jnp.concatenate([agga, aggb], axis=1)
    s = dinv * agg + (dinv * dinv) * xw_ref[...] + b_ref[...]
    h = jnp.maximum(s, 0.0)
    xw2 = jnp.dot(h, w_ref[...], preferred_element_type=jnp.float32)
    xw2_ref[...] = xw2
    y2_ref[...] = (xw2 * dinv).astype(jnp.bfloat16)


def _layer(aggpa, aggpb, xw1, degp, b1, w2, p64):
    d_in = xw1.shape[1]
    d_out = w2.shape[1]
    return pl.pallas_call(
        _layer_body,
        grid=(_G,),
        in_specs=[
            pl.BlockSpec((NUM_CORES, _R, D_OUT), lambda i: (0, i, 0)),
            pl.BlockSpec((NUM_CORES, _R, D_OUT), lambda i: (0, i, 0)),
            pl.BlockSpec((_R, d_in), lambda i: (i, 0)),
            pl.BlockSpec((NUM_CORES, _R, 16), lambda i: (0, i, 0)),
            pl.BlockSpec((1, d_in), lambda i: (0, 0)),
            pl.BlockSpec((d_in, d_out), lambda i: (0, 0)),
            pl.BlockSpec((D_OUT, D_OUT), lambda i: (0, 0)),
        ],
        out_specs=[
            pl.BlockSpec((_R, d_out), lambda i: (i, 0)),
            pl.BlockSpec((_R, d_out), lambda i: (i, 0)),
        ],
        out_shape=[
            jax.ShapeDtypeStruct((N, d_out), jnp.float32),
            jax.ShapeDtypeStruct((N, d_out), jnp.bfloat16),
        ],
    )(aggpa, aggpb, xw1, degp, b1, w2, p64)


def _final_body(aggp_ref, xw_ref, degp_ref, b_ref, p_ref, o_ref):
    dinv = _dinv_of(degp_ref)
    agg = jnp.dot(aggp_ref[0] + aggp_ref[1], p_ref[...],
                  preferred_element_type=jnp.float32)
    o_ref[...] = dinv * agg + (dinv * dinv) * xw_ref[...] + b_ref[...]


def _final(aggp, xw2, degp, b2, p64):
    d = xw2.shape[1]
    return pl.pallas_call(
        _final_body,
        grid=(_G,),
        in_specs=[
            pl.BlockSpec((NUM_CORES, _R, d), lambda i: (0, i, 0)),
            pl.BlockSpec((_R, d), lambda i: (i, 0)),
            pl.BlockSpec((NUM_CORES, _R, 16), lambda i: (0, i, 0)),
            pl.BlockSpec((1, d), lambda i: (0, 0)),
            pl.BlockSpec((D_OUT, D_OUT), lambda i: (0, 0)),
        ],
        out_specs=pl.BlockSpec((_R, d), lambda i: (i, 0)),
        out_shape=jax.ShapeDtypeStruct((N, d), jnp.float32),
    )(aggp, xw2, degp, b2, p64)


def kernel(x, edge_index, edge_attr, W1, b1, W2, b2):
    row = edge_index[0].reshape(NUM_TILES, NCH, K)
    col = edge_index[1].reshape(NUM_TILES, NCH, K)
    ew = edge_attr.reshape(NUM_TILES, NCH, K)
    b1r = b1.reshape(1, D_HID)
    b2r = b2.reshape(1, D_OUT)

    p64 = jnp.asarray(_P64)

    degp = _deg_kernel(col, ew)
    xw1, y1a, y1b = _matmul_prescale(x, W1, degp)
    aggp1a = _msg_kernel_64(y1a, row, col, ew)
    aggp1b = _msg_kernel_64(y1b, row, col, ew)
    xw2, y2 = _layer(aggp1a, aggp1b, xw1, degp, b1r, W2, p64)
    aggp2 = _msg_kernel_64(y2, row, col, ew)
    out = _final(aggp2, xw2, degp, b2r, p64)
    return out


# speedup vs baseline: 1.0160x; 1.0160x over previous
import functools

import jax
import jax.numpy as jnp
import numpy as np
from jax import lax
from jax.experimental import pallas as pl
from jax.experimental.pallas import tpu as pltpu
from jax.experimental.pallas import tpu_sc as plsc

N = 10000
E = 320000
D_IN = 128
D_HID = 128
D_OUT = 64

NUM_CORES = 2
NUM_SUBCORES = 16
NUM_TILES = NUM_CORES * NUM_SUBCORES
EPT = E // NUM_TILES
K = 80
NCH = EPT // K
RPT = N // NUM_SUBCORES
ZR = 125
WB = 624
WB_TAIL = N - NUM_SUBCORES * WB
NBUF = 4

_MESH = plsc.VectorSubcoreMesh(core_axis_name="c", subcore_axis_name="s")
_SC_PARAMS = pltpu.CompilerParams(use_tc_tiling_on_sc=False,
                                  needs_layout_passes=False)

_Q64 = np.empty(64, np.int64)
for _h in range(2):
    for _u in range(16):
        _Q64[32 * _h + _u] = 32 * _h + 2 * _u
        _Q64[32 * _h + 16 + _u] = 32 * _h + 2 * _u + 1
_P64 = np.zeros((64, 64), np.float32)
_P64[np.arange(64), _Q64] = 1.0


def _zero_acc(zbuf, acc, sid, d):
    def _zrow(i, carry):
        for j in range(d // 16):
            zbuf[i, pl.ds(j * 16, 16)] = jnp.zeros((16,), jnp.float32)
        return carry

    lax.fori_loop(0, ZR, _zrow, 0)
    for z in range(RPT // ZR):
        pltpu.sync_copy(zbuf, acc.at[pl.ds(sid * RPT + z * ZR, ZR)])


def _deg_body(col_hbm, ew_hbm, out_hbm, cidx, ewv, rows, zbuf, acc, sem):
    cid = lax.axis_index("c")
    sid = lax.axis_index("s")
    wid = cid * NUM_SUBCORES + sid

    _zero_acc(zbuf, acc, sid, 16)
    plsc.subcore_barrier()

    pltpu.sync_copy(col_hbm.at[wid], cidx)
    pltpu.sync_copy(ew_hbm.at[wid], ewv)

    def chunk(i, carry):
        def fill(g, c2):
            wv = ewv[i, pl.ds(g * 16, 16)]
            for j in range(16):
                rows[g * 16 + j, :] = jnp.full((16,), wv[j], jnp.float32)
            return c2

        lax.fori_loop(0, K // 16, fill, 0)
        pltpu.sync_copy(rows, acc.at[cidx.at[i]], add=True)
        return carry

    lax.fori_loop(0, NCH, chunk, 0)
    plsc.subcore_barrier()
    pltpu.sync_copy(acc.at[pl.ds(sid * WB, WB)],
                    out_hbm.at[cid, pl.ds(sid * WB, WB)])

    @pl.when(sid == 0)
    def _tail():
        pltpu.sync_copy(acc.at[pl.ds(NUM_SUBCORES * WB, WB_TAIL)],
                        out_hbm.at[cid, pl.ds(NUM_SUBCORES * WB, WB_TAIL)])


_deg_kernel = functools.partial(
    pl.kernel,
    out_type=jax.ShapeDtypeStruct((NUM_CORES, N, 16), jnp.float32),
    mesh=_MESH,
    scratch_types=[
        pltpu.VMEM((NCH, K), jnp.int32),
        pltpu.VMEM((NCH, K), jnp.float32),
        pltpu.VMEM((K, 16), jnp.float32),
        pltpu.VMEM((ZR, 16), jnp.float32),
        pltpu.VMEM_SHARED((N, 16), jnp.float32),
        pltpu.SemaphoreType.DMA,
    ],
    compiler_params=_SC_PARAMS,
)(_deg_body)


def _msg_body(d, y_hbm, row_hbm, col_hbm, ew_hbm, out_hbm,
              ridx, cidx, ewv, bufs, fbufs, zbuf, acc, gsems, ssems):
    cid = lax.axis_index("c")
    sid = lax.axis_index("s")
    wid = cid * NUM_SUBCORES + sid

    _zero_acc(zbuf, acc, sid, d)
    plsc.subcore_barrier()

    pltpu.sync_copy(row_hbm.at[wid], ridx)
    pltpu.sync_copy(col_hbm.at[wid], cidx)
    pltpu.sync_copy(ew_hbm.at[wid], ewv)

    def gather(i, q):
        return pltpu.make_async_copy(y_hbm.at[ridx.at[i]], bufs[q], gsems[q])

    def scatter(i, q):
        return pltpu.make_async_copy(fbufs[q], acc.at[cidx.at[i]], ssems[q])

    def scale(i, q):
        buf = bufs[q]
        fbuf = fbufs[q]

        def body(g, c2):
            wv = ewv[i, pl.ds(g * 16, 16)]
            for u in range(16):
                e = g * 16 + u
                w = wv[u]
                for h in range(d // 32):
                    x32 = buf[e, pl.ds(h * 32, 32)]
                    a, b = plsc.unpack(
                        x32, format=plsc.PackFormat.INTERLEAVED)
                    fbuf[e, pl.ds(h * 32, 16)] = a * w
                    fbuf[e, pl.ds(h * 32 + 16, 16)] = b * w
            return c2

        lax.fori_loop(0, K // 16, body, 0)

    pltpu.make_async_copy(zbuf.at[pl.ds(0, K)], acc.at[cidx.at[0]],
                          ssems[NBUF - 1]).start(add=True)
    for i in range(NBUF - 1):
        gather(i, i).start()

    def block(i, q, refill):
        gather(i, q).wait()
        scale(i, q)
        scatter(i, q).start(add=True)
        scatter(i - 1, (q - 1) % NBUF).wait()
        if refill:
            gather(i + NBUF - 1, (q - 1) % NBUF).start()

    def quad(t, carry):
        i0 = NBUF * t
        for q in range(NBUF):
            block(i0 + q, q, True)
        return carry

    nloop = (NCH - (NBUF + 1)) // NBUF
    lax.fori_loop(0, nloop, quad, 0)
    for i in range(NBUF * nloop, NCH):
        block(i, i % NBUF, i + NBUF - 1 < NCH)
    scatter(NCH - 1, (NCH - 1) % NBUF).wait()
    plsc.subcore_barrier()
    pltpu.sync_copy(acc.at[pl.ds(sid * WB, WB)],
                    out_hbm.at[cid, pl.ds(sid * WB, WB)])

    @pl.when(sid == 0)
    def _tail():
        pltpu.sync_copy(acc.at[pl.ds(NUM_SUBCORES * WB, WB_TAIL)],
                        out_hbm.at[cid, pl.ds(NUM_SUBCORES * WB, WB_TAIL)])


def _make_msg_kernel(d):
    return functools.partial(
        pl.kernel,
        out_type=jax.ShapeDtypeStruct((NUM_CORES, N, d), jnp.float32),
        mesh=_MESH,
        scratch_types=[
            pltpu.VMEM((NCH, K), jnp.int32),
            pltpu.VMEM((NCH, K), jnp.int32),
            pltpu.VMEM((NCH, K), jnp.float32),
            [pltpu.VMEM((K, d), jnp.bfloat16) for _ in range(NBUF)],
            [pltpu.VMEM((K, d), jnp.float32) for _ in range(NBUF)],
            pltpu.VMEM((ZR, d), jnp.float32),
            pltpu.VMEM_SHARED((N, d), jnp.float32),
            [pltpu.SemaphoreType.DMA for _ in range(NBUF)],
            [pltpu.SemaphoreType.DMA for _ in range(NBUF)],
        ],
        compiler_params=_SC_PARAMS,
    )(functools.partial(_msg_body, d))


_msg_kernel_64 = _make_msg_kernel(D_OUT)


_R = 1000
_G = N // _R


def _dinv_of(degp_ref):
    deg = degp_ref[0, :, :1] + degp_ref[1, :, :1] + 1.0
    return lax.rsqrt(deg)


def _mm_body(x_ref, w_ref, degp_ref, xw_ref, ya_ref, yb_ref):
    xw = jnp.dot(x_ref[...], w_ref[...], preferred_element_type=jnp.float32)
    xw_ref[...] = xw
    y = (xw * _dinv_of(degp_ref)).astype(jnp.bfloat16)
    ya_ref[...] = y[:, :D_OUT]
    yb_ref[...] = y[:, D_OUT:]


def _matmul_prescale(x, w, degp):
    return pl.pallas_call(
        _mm_body,
        grid=(_G,),
        in_specs=[
            pl.BlockSpec((_R, x.shape[1]), lambda i: (i, 0)),
            pl.BlockSpec(w.shape, lambda i: (0, 0)),
            pl.BlockSpec((NUM_CORES, _R, 16), lambda i: (0, i, 0)),
        ],
        out_specs=[
            pl.BlockSpec((_R, w.shape[1]), lambda i: (i, 0)),
            pl.BlockSpec((_R, D_OUT), lambda i: (i, 0)),
            pl.BlockSpec((_R, D_OUT), lambda i: (i, 0)),
        ],
        out_shape=[
            jax.ShapeDtypeStruct((x.shape[0], w.shape[1]), jnp.float32),
            jax.ShapeDtypeStruct((N, D_OUT), jnp.bfloat16),
            jax.ShapeDtypeStruct((N, D_OUT), jnp.bfloat16),
        ],
    )(x, w, degp)


def _layer_body(aggpa_ref, aggpb_ref, xw_ref, degp_ref, b_ref, w_ref,
                p_ref, xw2_ref, y2_ref):
    dinv = _dinv_of(degp_ref)
    p = p_ref[...]
    agga = jnp.dot(aggpa_ref[0] + aggpa_ref[1], p,
                   preferred_element_type=jnp.float32)
    aggb = jnp.dot(aggpb_ref[0] + aggpb_ref[1], p,
                   preferred_element_type=jnp.float32)
    agg = jnp.concatenate([agga, aggb], axis=1)
    s = dinv * agg + (dinv * dinv) * xw_ref[...] + b_ref[...]
    h = jnp.maximum(s, 0.0)
    xw2 = jnp.dot(h, w_ref[...], preferred_element_type=jnp.float32)
    xw2_ref[...] = xw2
    y2_ref[...] = (xw2 * dinv).astype(jnp.bfloat16)


def _layer(aggpa, aggpb, xw1, degp, b1, w2, p64):
    d_in = xw1.shape[1]
    d_out = w2.shape[1]
    return pl.pallas_call(
        _layer_body,
        grid=(_G,),
        in_specs=[
            pl.BlockSpec((NUM_CORES, _R, D_OUT), lambda i: (0, i, 0)),
            pl.BlockSpec((NUM_CORES, _R, D_OUT), lambda i: (0, i, 0)),
            pl.BlockSpec((_R, d_in), lambda i: (i, 0)),
            pl.BlockSpec((NUM_CORES, _R, 16), lambda i: (0, i, 0)),
            pl.BlockSpec((1, d_in), lambda i: (0, 0)),
            pl.BlockSpec((d_in, d_out), lambda i: (0, 0)),
            pl.BlockSpec((D_OUT, D_OUT), lambda i: (0, 0)),
        ],
        out_specs=[
            pl.BlockSpec((_R, d_out), lambda i: (i, 0)),
            pl.BlockSpec((_R, d_out), lambda i: (i, 0)),
        ],
        out_shape=[
            jax.ShapeDtypeStruct((N, d_out), jnp.float32),
            jax.ShapeDtypeStruct((N, d_out), jnp.bfloat16),
        ],
    )(aggpa, aggpb, xw1, degp, b1, w2, p64)


def _final_body(aggp_ref, xw_ref, degp_ref, b_ref, p_ref, o_ref):
    dinv = _dinv_of(degp_ref)
    agg = jnp.dot(aggp_ref[0] + aggp_ref[1], p_ref[...],
                  preferred_element_type=jnp.float32)
    o_ref[...] = dinv * agg + (dinv * dinv) * xw_ref[...] + b_ref[...]


def _final(aggp, xw2, degp, b2, p64):
    d = xw2.shape[1]
    return pl.pallas_call(
        _final_body,
        grid=(_G,),
        in_specs=[
            pl.BlockSpec((NUM_CORES, _R, d), lambda i: (0, i, 0)),
            pl.BlockSpec((_R, d), lambda i: (i, 0)),
            pl.BlockSpec((NUM_CORES, _R, 16), lambda i: (0, i, 0)),
            pl.BlockSpec((1, d), lambda i: (0, 0)),
            pl.BlockSpec((D_OUT, D_OUT), lambda i: (0, 0)),
        ],
        out_specs=pl.BlockSpec((_R, d), lambda i: (i, 0)),
        out_shape=jax.ShapeDtypeStruct((N, d), jnp.float32),
    )(aggp, xw2, degp, b2, p64)


def kernel(x, edge_index, edge_attr, W1, b1, W2, b2):
    row = edge_index[0].reshape(NUM_TILES, NCH, K)
    col = edge_index[1].reshape(NUM_TILES, NCH, K)
    ew = edge_attr.reshape(NUM_TILES, NCH, K)
    b1r = b1.reshape(1, D_HID)
    b2r = b2.reshape(1, D_OUT)

    p64 = jnp.asarray(_P64)

    degp = _deg_kernel(col, ew)
    xw1, y1a, y1b = _matmul_prescale(x, W1, degp)
    aggp1a = _msg_kernel_64(y1a, row, col, ew)
    aggp1b = _msg_kernel_64(y1b, row, col, ew)
    xw2, y2 = _layer(aggp1a, aggp1b, xw1, degp, b1r, W2, p64)
    aggp2 = _msg_kernel_64(y2, row, col, ew)
    out = _final(aggp2, xw2, degp, b2r, p64)
    return out
